# Initial kernel scaffold; baseline (speedup 1.0000x reference)
import jax
import jax.numpy as jnp
from jax.experimental import pallas as pl


def kernel(input_ids, token_type_ids, word_table, type_table, gamma, beta):
    raise NotImplementedError("write your pallas kernel here")


# speedup vs baseline: 1.0409x; 1.0409x over previous
import functools

import jax
import jax.numpy as jnp
from jax import lax
from jax.experimental import pallas as pl
from jax.experimental.pallas import tpu as pltpu
from jax.experimental.pallas import tpu_sc as plsc

_VOCAB = 100000
_HID = 768
_B = 4
_S = 2048
_EPS = 1e-12

_NC = 2
_NS = 16
_NW = _NC * _NS
_TOK = _B * _S
_PER_W = _TOK // _NW
_CHUNK = 64
_NCH = _PER_W // _CHUNK


def _sc_gather_body(table_hbm, idx_hbm, out_hbm, idx_v, rows_v, sem0, sem1):
    wid = lax.axis_index("s") * _NC + lax.axis_index("c")
    base = wid * _PER_W
    pltpu.sync_copy(idx_hbm.at[wid], idx_v)
    sems = (sem0, sem1)
    cp0 = pltpu.async_copy(table_hbm.at[idx_v.at[0]], rows_v.at[0], sems[0])
    copies = [cp0, None]
    for c in range(_NCH):
        b = c % 2
        if c + 1 < _NCH:
            nb = (c + 1) % 2
            copies[nb] = pltpu.async_copy(
                table_hbm.at[idx_v.at[c + 1]], rows_v.at[nb], sems[nb]
            )
        copies[b].wait()
        pltpu.sync_copy(rows_v.at[b], out_hbm.at[pl.ds(base + c * _CHUNK, _CHUNK)])


@jax.jit
def _sc_gather(word_table, idx3):
    mesh = plsc.VectorSubcoreMesh(
        core_axis_name="c", subcore_axis_name="s", num_cores=_NC, num_subcores=_NS
    )
    return pl.kernel(
        _sc_gather_body,
        out_type=jax.ShapeDtypeStruct((_TOK, _HID), jnp.float32),
        mesh=mesh,
        scratch_types=[
            pltpu.VMEM((_NCH, _CHUNK), jnp.int32),
            pltpu.VMEM((2, _CHUNK, _HID), jnp.float32),
            pltpu.SemaphoreType.DMA,
            pltpu.SemaphoreType.DMA,
        ],
    )(word_table, idx3)


_ROWS_BLK = 256


def _tc_fuse_body(rows_ref, tt_ref, type_ref, gamma_ref, beta_ref, out_ref):
    i = pl.program_id(0)
    rows = rows_ref[...]
    half = _HID // 2
    s0 = (i * _ROWS_BLK) % _S
    pos = (s0 + lax.broadcasted_iota(jnp.int32, (_ROWS_BLK, 1), 0)).astype(jnp.float32)
    h_idx = lax.broadcasted_iota(jnp.int32, (1, _HID), 1)
    h_mod = jnp.where(h_idx < half, h_idx, h_idx - half).astype(jnp.float32)
    inv_freq = jnp.exp(h_mod * (-2.0 * jnp.log(10000.0) / _HID))
    ang = pos * inv_freq
    pos_emb = jnp.where(h_idx < half, jnp.sin(ang), jnp.cos(ang))
    tt = tt_ref[...]
    type_emb = jnp.where(tt == 0, type_ref[0:1, :], type_ref[1:2, :])
    e = rows + pos_emb + type_emb
    mean = jnp.mean(e, axis=1, keepdims=True)
    d = e - mean
    var = jnp.mean(d * d, axis=1, keepdims=True)
    normed = d * lax.rsqrt(var + _EPS)
    out_ref[...] = normed * gamma_ref[...] + beta_ref[...]


@jax.jit
def _tc_fuse(rows, tt2, type_table, gamma2, beta2):
    grid = (_TOK // _ROWS_BLK,)
    return pl.pallas_call(
        _tc_fuse_body,
        grid=grid,
        in_specs=[
            pl.BlockSpec((_ROWS_BLK, _HID), lambda i: (i, 0)),
            pl.BlockSpec((_ROWS_BLK, 1), lambda i: (i, 0)),
            pl.BlockSpec((2, _HID), lambda i: (0, 0)),
            pl.BlockSpec((1, _HID), lambda i: (0, 0)),
            pl.BlockSpec((1, _HID), lambda i: (0, 0)),
        ],
        out_specs=pl.BlockSpec((_ROWS_BLK, _HID), lambda i: (i, 0)),
        out_shape=jax.ShapeDtypeStruct((_TOK, _HID), jnp.float32),
    )(rows, tt2, type_table, gamma2, beta2)


def kernel(input_ids, token_type_ids, word_table, type_table, gamma, beta):
    idx3 = input_ids.reshape(_NW, _NCH, _CHUNK)
    rows = _sc_gather(word_table, idx3)
    tt2 = token_type_ids.reshape(_TOK, 1)
    out = _tc_fuse(rows, tt2, type_table, gamma.reshape(1, _HID), beta.reshape(1, _HID))
    return out.reshape(_B, _S, _HID)


# speedup vs baseline: 1.3471x; 1.2942x over previous
import functools

import jax
import jax.numpy as jnp
from jax import lax
from jax.experimental import pallas as pl
from jax.experimental.pallas import tpu as pltpu
from jax.experimental.pallas import tpu_sc as plsc

_VOCAB = 100000
_HID = 768
_B = 4
_S = 2048
_EPS = 1e-12

_NC = 2
_NS = 16
_NW = _NC * _NS
_TOK = _B * _S
_PER_W = _TOK // _NW
_CHUNK = 64
_NCH = _PER_W // _CHUNK


def _sc_gather_body(table_hbm, idx_hbm, out_hbm, idx_v, rows_v, sem0, sem1):
    wid = lax.axis_index("s") * _NC + lax.axis_index("c")
    base = wid * _PER_W
    pltpu.sync_copy(idx_hbm.at[wid], idx_v)
    sems = (sem0, sem1)
    cp0 = pltpu.async_copy(table_hbm.at[idx_v.at[0]], rows_v.at[0], sems[0])
    copies = [cp0, None]
    for c in range(_NCH):
        b = c % 2
        if c + 1 < _NCH:
            nb = (c + 1) % 2
            copies[nb] = pltpu.async_copy(
                table_hbm.at[idx_v.at[c + 1]], rows_v.at[nb], sems[nb]
            )
        copies[b].wait()
        pltpu.sync_copy(rows_v.at[b], out_hbm.at[pl.ds(base + c * _CHUNK, _CHUNK)])


@jax.jit
def _sc_gather(word_table, idx3):
    mesh = plsc.VectorSubcoreMesh(
        core_axis_name="c", subcore_axis_name="s", num_cores=_NC, num_subcores=_NS
    )
    return pl.kernel(
        _sc_gather_body,
        out_type=jax.ShapeDtypeStruct((_TOK, _HID), jnp.float32),
        mesh=mesh,
        scratch_types=[
            pltpu.VMEM((_NCH, _CHUNK), jnp.int32),
            pltpu.VMEM((2, _CHUNK, _HID), jnp.float32),
            pltpu.SemaphoreType.DMA,
            pltpu.SemaphoreType.DMA,
        ],
    )(word_table, idx3)


_ROWS_BLK = 256
_SBLK = _S // _ROWS_BLK


def _tc_fuse_body(rows_ref, tt_ref, type_ref, gamma_ref, beta_ref, out_ref, pos_ref):
    i = pl.program_id(0)
    j = pl.program_id(1)
    half = _HID // 2

    @pl.when(j == 0)
    def _():
        s0 = i * _ROWS_BLK
        pos = (s0 + lax.broadcasted_iota(jnp.int32, (_ROWS_BLK, 1), 0)).astype(
            jnp.float32
        )
        h_idx = lax.broadcasted_iota(jnp.int32, (1, _HID), 1)
        h_mod = jnp.where(h_idx < half, h_idx, h_idx - half).astype(jnp.float32)
        inv_freq = jnp.exp(h_mod * (-2.0 * jnp.log(10000.0) / _HID))
        ang = pos * inv_freq
        pos_ref[...] = jnp.where(h_idx < half, jnp.sin(ang), jnp.cos(ang))

    rows = rows_ref[...]
    tt = tt_ref[...]
    type_emb = jnp.where(tt == 0, type_ref[0:1, :], type_ref[1:2, :])
    e = rows + pos_ref[...] + type_emb
    mean = jnp.mean(e, axis=1, keepdims=True)
    d = e - mean
    var = jnp.mean(d * d, axis=1, keepdims=True)
    normed = d * lax.rsqrt(var + _EPS)
    out_ref[...] = normed * gamma_ref[...] + beta_ref[...]


@jax.jit
def _tc_fuse(rows, tt2, type_table, gamma2, beta2):
    grid = (_SBLK, _B)
    rows_map = lambda i, j: (j * _SBLK + i, 0)
    return pl.pallas_call(
        _tc_fuse_body,
        grid=grid,
        in_specs=[
            pl.BlockSpec((_ROWS_BLK, _HID), rows_map),
            pl.BlockSpec((_ROWS_BLK, 1), rows_map),
            pl.BlockSpec((2, _HID), lambda i, j: (0, 0)),
            pl.BlockSpec((1, _HID), lambda i, j: (0, 0)),
            pl.BlockSpec((1, _HID), lambda i, j: (0, 0)),
        ],
        out_specs=pl.BlockSpec((_ROWS_BLK, _HID), rows_map),
        out_shape=jax.ShapeDtypeStruct((_TOK, _HID), jnp.float32),
        scratch_shapes=[pltpu.VMEM((_ROWS_BLK, _HID), jnp.float32)],
    )(rows, tt2, type_table, gamma2, beta2)


def kernel(input_ids, token_type_ids, word_table, type_table, gamma, beta):
    idx3 = input_ids.reshape(_NW, _NCH, _CHUNK)
    rows = _sc_gather(word_table, idx3)
    tt2 = token_type_ids.reshape(_TOK, 1)
    out = _tc_fuse(rows, tt2, type_table, gamma.reshape(1, _HID), beta.reshape(1, _HID))
    return out.reshape(_B, _S, _HID)


# speedup vs baseline: 1.5978x; 1.1861x over previous
import functools

import jax
import jax.numpy as jnp
from jax import lax
from jax.experimental import pallas as pl
from jax.experimental.pallas import tpu as pltpu
from jax.experimental.pallas import tpu_sc as plsc

_VOCAB = 100000
_HID = 768
_B = 4
_S = 2048
_EPS = 1e-12

_NC = 2
_NS = 16
_NW = _NC * _NS
_TOK = _B * _S
_PER_W = _TOK // _NW
_CHUNK = 64
_NCH = _PER_W // _CHUNK


def _sc_gather_body(table_hbm, idx_hbm, out_hbm, idx_v, rows_v, sem0, sem1):
    wid = lax.axis_index("s") * _NC + lax.axis_index("c")
    base = wid * _PER_W
    pltpu.sync_copy(idx_hbm.at[wid], idx_v)
    sems = (sem0, sem1)
    cp0 = pltpu.async_copy(table_hbm.at[idx_v.at[0]], rows_v.at[0], sems[0])
    copies = [cp0, None]
    for c in range(_NCH):
        b = c % 2
        if c + 1 < _NCH:
            nb = (c + 1) % 2
            copies[nb] = pltpu.async_copy(
                table_hbm.at[idx_v.at[c + 1]], rows_v.at[nb], sems[nb]
            )
        copies[b].wait()
        pltpu.sync_copy(rows_v.at[b], out_hbm.at[pl.ds(base + c * _CHUNK, _CHUNK)])


@jax.jit
def _sc_gather(word_table, idx3):
    mesh = plsc.VectorSubcoreMesh(
        core_axis_name="c", subcore_axis_name="s", num_cores=_NC, num_subcores=_NS
    )
    return pl.kernel(
        _sc_gather_body,
        out_type=jax.ShapeDtypeStruct((_TOK, _HID), jnp.float32),
        mesh=mesh,
        scratch_types=[
            pltpu.VMEM((_NCH, _CHUNK), jnp.int32),
            pltpu.VMEM((2, _CHUNK, _HID), jnp.float32),
            pltpu.SemaphoreType.DMA,
            pltpu.SemaphoreType.DMA,
        ],
    )(word_table, idx3)


_ROWS_BLK = 256
_SBLK = _S // _ROWS_BLK


def _pos_body(out_ref):
    i = pl.program_id(0)
    half = _HID // 2
    s0 = i * _ROWS_BLK
    pos = (s0 + lax.broadcasted_iota(jnp.int32, (_ROWS_BLK, 1), 0)).astype(jnp.float32)
    h_idx = lax.broadcasted_iota(jnp.int32, (1, _HID), 1)
    h_mod = jnp.where(h_idx < half, h_idx, h_idx - half).astype(jnp.float32)
    inv_freq = jnp.exp(h_mod * (-2.0 * jnp.log(10000.0) / _HID))
    ang = pos * inv_freq
    out_ref[...] = jnp.where(h_idx < half, jnp.sin(ang), jnp.cos(ang))


def _pos_table():
    return pl.pallas_call(
        _pos_body,
        grid=(_SBLK,),
        out_specs=pl.BlockSpec((_ROWS_BLK, _HID), lambda i: (i, 0)),
        out_shape=jax.ShapeDtypeStruct((_S, _HID), jnp.float32),
    )()


def _tc_fuse_body(rows_ref, pos_ref, tt_ref, type_ref, gamma_ref, beta_ref, out_ref):
    rows = rows_ref[...]
    tt = tt_ref[...]
    type_emb = jnp.where(tt == 0, type_ref[0:1, :], type_ref[1:2, :])
    e = rows + pos_ref[...] + type_emb
    mean = jnp.mean(e, axis=1, keepdims=True)
    d = e - mean
    var = jnp.mean(d * d, axis=1, keepdims=True)
    normed = d * lax.rsqrt(var + _EPS)
    out_ref[...] = normed * gamma_ref[...] + beta_ref[...]


def _tc_fuse(rows, pos, tt2, type_table, gamma2, beta2):
    grid = (_SBLK, _B)
    rows_map = lambda i, j: (j * _SBLK + i, 0)
    return pl.pallas_call(
        _tc_fuse_body,
        grid=grid,
        in_specs=[
            pl.BlockSpec((_ROWS_BLK, _HID), rows_map),
            pl.BlockSpec((_ROWS_BLK, _HID), lambda i, j: (i, 0)),
            pl.BlockSpec((_ROWS_BLK, 1), rows_map),
            pl.BlockSpec((2, _HID), lambda i, j: (0, 0)),
            pl.BlockSpec((1, _HID), lambda i, j: (0, 0)),
            pl.BlockSpec((1, _HID), lambda i, j: (0, 0)),
        ],
        out_specs=pl.BlockSpec((_ROWS_BLK, _HID), rows_map),
        out_shape=jax.ShapeDtypeStruct((_TOK, _HID), jnp.float32),
    )(rows, pos, tt2, type_table, gamma2, beta2)


def kernel(input_ids, token_type_ids, word_table, type_table, gamma, beta):
    idx3 = input_ids.reshape(_NW, _NCH, _CHUNK)
    rows = _sc_gather(word_table, idx3)
    pos = _pos_table()
    tt2 = token_type_ids.reshape(_TOK, 1)
    out = _tc_fuse(
        rows, pos, tt2, type_table, gamma.reshape(1, _HID), beta.reshape(1, _HID)
    )
    return out.reshape(_B, _S, _HID)


# speedup vs baseline: 1.7966x; 1.1244x over previous
import functools

import jax
import jax.numpy as jnp
from jax import lax
from jax.experimental import pallas as pl
from jax.experimental.pallas import tpu as pltpu
from jax.experimental.pallas import tpu_sc as plsc

_VOCAB = 100000
_HID = 768
_B = 4
_S = 2048
_EPS = 1e-12

_NC = 2
_NS = 16
_NW = _NC * _NS
_TOK = _B * _S
_PER_W = _TOK // _NW
_CHUNK = 64
_NCH = _PER_W // _CHUNK


def _sc_gather_body(table_hbm, idx_hbm, out_hbm, idx_v, rows_v, sem0, sem1):
    wid = lax.axis_index("s") * _NC + lax.axis_index("c")
    base = wid * _PER_W
    pltpu.sync_copy(idx_hbm.at[wid], idx_v)
    sems = (sem0, sem1)
    cp0 = pltpu.async_copy(table_hbm.at[idx_v.at[0]], rows_v.at[0], sems[0])
    copies = [cp0, None]
    for c in range(_NCH):
        b = c % 2
        if c + 1 < _NCH:
            nb = (c + 1) % 2
            copies[nb] = pltpu.async_copy(
                table_hbm.at[idx_v.at[c + 1]], rows_v.at[nb], sems[nb]
            )
        copies[b].wait()
        pltpu.sync_copy(rows_v.at[b], out_hbm.at[pl.ds(base + c * _CHUNK, _CHUNK)])


@jax.jit
def _sc_gather(word_table, idx3):
    mesh = plsc.VectorSubcoreMesh(
        core_axis_name="c", subcore_axis_name="s", num_cores=_NC, num_subcores=_NS
    )
    return pl.kernel(
        _sc_gather_body,
        out_type=jax.ShapeDtypeStruct((_TOK, _HID), jnp.float32),
        mesh=mesh,
        scratch_types=[
            pltpu.VMEM((_NCH, _CHUNK), jnp.int32),
            pltpu.VMEM((2, _CHUNK, _HID), jnp.float32),
            pltpu.SemaphoreType.DMA,
            pltpu.SemaphoreType.DMA,
        ],
    )(word_table, idx3)


_ROWS_BLK = 512
_SBLK = _S // _ROWS_BLK


def _pos_body(out_ref):
    i = pl.program_id(0)
    half = _HID // 2
    s0 = i * _ROWS_BLK
    pos = (s0 + lax.broadcasted_iota(jnp.int32, (_ROWS_BLK, 1), 0)).astype(jnp.float32)
    h_idx = lax.broadcasted_iota(jnp.int32, (1, _HID), 1)
    h_mod = jnp.where(h_idx < half, h_idx, h_idx - half).astype(jnp.float32)
    inv_freq = jnp.exp(h_mod * (-2.0 * jnp.log(10000.0) / _HID))
    shift = jnp.where(h_idx < half, 0.0, 0.5 * jnp.pi).astype(jnp.float32)
    out_ref[...] = jnp.sin(pos * inv_freq + shift)


def _pos_table():
    return pl.pallas_call(
        _pos_body,
        grid=(_SBLK,),
        out_specs=pl.BlockSpec((_ROWS_BLK, _HID), lambda i: (i, 0)),
        out_shape=jax.ShapeDtypeStruct((_S, _HID), jnp.float32),
    )()


def _tc_fuse_body(rows_ref, pos_ref, tt_ref, type_ref, gamma_ref, beta_ref, out_ref):
    rows = rows_ref[...]
    tt = tt_ref[...]
    type_emb = jnp.where(tt == 0, type_ref[0:1, :], type_ref[1:2, :])
    e = rows + pos_ref[...] + type_emb
    mean = jnp.mean(e, axis=1, keepdims=True)
    d = e - mean
    var = jnp.mean(d * d, axis=1, keepdims=True)
    normed = d * lax.rsqrt(var + _EPS)
    out_ref[...] = normed * gamma_ref[...] + beta_ref[...]


def _tc_fuse(rows, pos, tt2, type_table, gamma2, beta2):
    grid = (_SBLK, _B)
    rows_map = lambda i, j: (j * _SBLK + i, 0)
    return pl.pallas_call(
        _tc_fuse_body,
        grid=grid,
        in_specs=[
            pl.BlockSpec((_ROWS_BLK, _HID), rows_map),
            pl.BlockSpec((_ROWS_BLK, _HID), lambda i, j: (i, 0)),
            pl.BlockSpec((_ROWS_BLK, 1), rows_map),
            pl.BlockSpec((2, _HID), lambda i, j: (0, 0)),
            pl.BlockSpec((1, _HID), lambda i, j: (0, 0)),
            pl.BlockSpec((1, _HID), lambda i, j: (0, 0)),
        ],
        out_specs=pl.BlockSpec((_ROWS_BLK, _HID), rows_map),
        out_shape=jax.ShapeDtypeStruct((_TOK, _HID), jnp.float32),
    )(rows, pos, tt2, type_table, gamma2, beta2)


def kernel(input_ids, token_type_ids, word_table, type_table, gamma, beta):
    idx3 = input_ids.reshape(_NW, _NCH, _CHUNK)
    rows = _sc_gather(word_table, idx3)
    pos = _pos_table()
    tt2 = token_type_ids.reshape(_TOK, 1)
    out = _tc_fuse(
        rows, pos, tt2, type_table, gamma.reshape(1, _HID), beta.reshape(1, _HID)
    )
    return out.reshape(_B, _S, _HID)


# speedup vs baseline: 1.9278x; 1.0730x over previous
import functools

import jax
import jax.numpy as jnp
from jax import lax
from jax.experimental import pallas as pl
from jax.experimental.pallas import tpu as pltpu
from jax.experimental.pallas import tpu_sc as plsc

_VOCAB = 100000
_HID = 768
_B = 4
_S = 2048
_EPS = 1e-12

_NC = 2
_NS = 16
_NW = _NC * _NS
_TOK = _B * _S
_PER_W = _TOK // _NW
_CHUNK = 64
_NCH = _PER_W // _CHUNK


def _sc_gather_body(table_hbm, idx_hbm, out_hbm, idx_v, rows_v, sem0, sem1):
    wid = lax.axis_index("s") * _NC + lax.axis_index("c")
    base = wid * _PER_W
    pltpu.sync_copy(idx_hbm.at[wid], idx_v)
    sems = (sem0, sem1)
    cp0 = pltpu.async_copy(table_hbm.at[idx_v.at[0]], rows_v.at[0], sems[0])
    copies = [cp0, None]
    for c in range(_NCH):
        b = c % 2
        if c + 1 < _NCH:
            nb = (c + 1) % 2
            copies[nb] = pltpu.async_copy(
                table_hbm.at[idx_v.at[c + 1]], rows_v.at[nb], sems[nb]
            )
        copies[b].wait()
        pltpu.sync_copy(rows_v.at[b], out_hbm.at[pl.ds(base + c * _CHUNK, _CHUNK)])


@jax.jit
def _sc_gather(word_table, idx3):
    mesh = plsc.VectorSubcoreMesh(
        core_axis_name="c", subcore_axis_name="s", num_cores=_NC, num_subcores=_NS
    )
    return pl.kernel(
        _sc_gather_body,
        out_type=jax.ShapeDtypeStruct((_TOK, _HID), jnp.float32),
        mesh=mesh,
        scratch_types=[
            pltpu.VMEM((_NCH, _CHUNK), jnp.int32),
            pltpu.VMEM((2, _CHUNK, _HID), jnp.float32),
            pltpu.SemaphoreType.DMA,
            pltpu.SemaphoreType.DMA,
        ],
    )(word_table, idx3)


_ROWS_BLK = 1024
_SBLK = _S // _ROWS_BLK


def _pos_body(out_ref):
    i = pl.program_id(0)
    half = _HID // 2
    s0 = i * _ROWS_BLK
    pos = (s0 + lax.broadcasted_iota(jnp.int32, (_ROWS_BLK, 1), 0)).astype(jnp.float32)
    h_idx = lax.broadcasted_iota(jnp.int32, (1, _HID), 1)
    h_mod = jnp.where(h_idx < half, h_idx, h_idx - half).astype(jnp.float32)
    inv_freq = jnp.exp(h_mod * (-2.0 * jnp.log(10000.0) / _HID))
    shift = jnp.where(h_idx < half, 0.0, 0.5 * jnp.pi).astype(jnp.float32)
    out_ref[...] = jnp.sin(pos * inv_freq + shift)


def _pos_table():
    return pl.pallas_call(
        _pos_body,
        grid=(_SBLK,),
        out_specs=pl.BlockSpec((_ROWS_BLK, _HID), lambda i: (i, 0)),
        out_shape=jax.ShapeDtypeStruct((_S, _HID), jnp.float32),
    )()


def _tc_fuse_body(rows_ref, pos_ref, tt_ref, type_ref, gamma_ref, beta_ref, out_ref):
    rows = rows_ref[...]
    ttf = jnp.reshape(tt_ref[0], (1, _ROWS_BLK)).T
    type_emb = type_ref[0:1, :] + ttf * (type_ref[1:2, :] - type_ref[0:1, :])
    e = rows + pos_ref[...] + type_emb
    mean = jnp.mean(e, axis=1, keepdims=True)
    d = e - mean
    var = jnp.mean(d * d, axis=1, keepdims=True)
    normed = d * lax.rsqrt(var + _EPS)
    out_ref[...] = normed * gamma_ref[...] + beta_ref[...]


def _tc_fuse(rows, pos, tt2, type_table, gamma2, beta2):
    grid = (_SBLK, _B)
    rows_map = lambda i, j: (j * _SBLK + i, 0)
    return pl.pallas_call(
        _tc_fuse_body,
        grid=grid,
        in_specs=[
            pl.BlockSpec((_ROWS_BLK, _HID), rows_map),
            pl.BlockSpec((_ROWS_BLK, _HID), lambda i, j: (i, 0)),
            pl.BlockSpec((1, 1, _ROWS_BLK), lambda i, j: (j * _SBLK + i, 0, 0)),
            pl.BlockSpec((2, _HID), lambda i, j: (0, 0)),
            pl.BlockSpec((1, _HID), lambda i, j: (0, 0)),
            pl.BlockSpec((1, _HID), lambda i, j: (0, 0)),
        ],
        out_specs=pl.BlockSpec((_ROWS_BLK, _HID), rows_map),
        out_shape=jax.ShapeDtypeStruct((_TOK, _HID), jnp.float32),
    )(rows, pos, tt2, type_table, gamma2, beta2)


def kernel(input_ids, token_type_ids, word_table, type_table, gamma, beta):
    idx3 = input_ids.reshape(_NW, _NCH, _CHUNK)
    rows = _sc_gather(word_table, idx3)
    pos = _pos_table()
    ttf = token_type_ids.astype(jnp.float32).reshape(_B * _SBLK, 1, _ROWS_BLK)
    out = _tc_fuse(
        rows, pos, ttf, type_table, gamma.reshape(1, _HID), beta.reshape(1, _HID)
    )
    return out.reshape(_B, _S, _HID)


# speedup vs baseline: 1.9652x; 1.0194x over previous
import functools

import jax
import jax.numpy as jnp
from jax import lax
from jax.experimental import pallas as pl
from jax.experimental.pallas import tpu as pltpu
from jax.experimental.pallas import tpu_sc as plsc

_VOCAB = 100000
_HID = 768
_B = 4
_S = 2048
_EPS = 1e-12

_NC = 2
_NS = 16
_NW = _NC * _NS
_TOK = _B * _S
_PER_W = _TOK // _NW
_CHUNK = 64
_NCH = _PER_W // _CHUNK


def _sc_gather_body(table_hbm, idx_hbm, out_hbm, idx_v, rows_v, sem0, sem1):
    wid = lax.axis_index("s") * _NC + lax.axis_index("c")
    base = wid * _PER_W
    pltpu.sync_copy(idx_hbm.at[wid], idx_v)
    sems = (sem0, sem1)
    cp0 = pltpu.async_copy(table_hbm.at[idx_v.at[0]], rows_v.at[0], sems[0])
    copies = [cp0, None]
    for c in range(_NCH):
        b = c % 2
        if c + 1 < _NCH:
            nb = (c + 1) % 2
            copies[nb] = pltpu.async_copy(
                table_hbm.at[idx_v.at[c + 1]], rows_v.at[nb], sems[nb]
            )
        copies[b].wait()
        pltpu.sync_copy(rows_v.at[b], out_hbm.at[pl.ds(base + c * _CHUNK, _CHUNK)])


@jax.jit
def _sc_gather(word_table, idx3):
    mesh = plsc.VectorSubcoreMesh(
        core_axis_name="c", subcore_axis_name="s", num_cores=_NC, num_subcores=_NS
    )
    return pl.kernel(
        _sc_gather_body,
        out_type=jax.ShapeDtypeStruct((_TOK, _HID), jnp.float32),
        mesh=mesh,
        scratch_types=[
            pltpu.VMEM((_NCH, _CHUNK), jnp.int32),
            pltpu.VMEM((2, _CHUNK, _HID), jnp.float32),
            pltpu.SemaphoreType.DMA,
            pltpu.SemaphoreType.DMA,
        ],
    )(word_table, idx3)


_ROWS_BLK = 1024
_SBLK = _S // _ROWS_BLK


def _pos_body(out_ref, prev_ref):
    i = pl.program_id(0)
    half = _HID // 2
    h_idx = lax.broadcasted_iota(jnp.int32, (1, _HID), 1)
    h_mod = jnp.where(h_idx < half, h_idx, h_idx - half).astype(jnp.float32)
    inv_freq = jnp.exp(h_mod * (-2.0 * jnp.log(10000.0) / _HID))

    @pl.when(i == 0)
    def _():
        pos = lax.broadcasted_iota(jnp.int32, (_ROWS_BLK, 1), 0).astype(jnp.float32)
        shift = jnp.where(h_idx < half, 0.0, 0.5 * jnp.pi).astype(jnp.float32)
        blk = jnp.sin(pos * inv_freq + shift)
        out_ref[...] = blk
        prev_ref[...] = blk

    @pl.when(i > 0)
    def _():
        rot_s = jnp.sin(_ROWS_BLK * inv_freq)
        rot_c = jnp.sin(_ROWS_BLK * inv_freq + 0.5 * jnp.pi)
        prev = prev_ref[...]
        partner = jnp.concatenate([prev[:, half:], prev[:, :half]], axis=1)
        sign = jnp.where(h_idx < half, 1.0, -1.0).astype(jnp.float32)
        blk = prev * rot_c + sign * partner * rot_s
        out_ref[...] = blk
        prev_ref[...] = blk


def _pos_table():
    return pl.pallas_call(
        _pos_body,
        grid=(_SBLK,),
        out_specs=pl.BlockSpec((_ROWS_BLK, _HID), lambda i: (i, 0)),
        out_shape=jax.ShapeDtypeStruct((_S, _HID), jnp.float32),
        scratch_shapes=[pltpu.VMEM((_ROWS_BLK, _HID), jnp.float32)],
    )()


def _tc_fuse_body(rows_ref, pos_ref, tt_ref, type_ref, gamma_ref, beta_ref, out_ref):
    rows = rows_ref[...]
    ttf = jnp.reshape(tt_ref[0], (1, _ROWS_BLK)).T
    type_emb = type_ref[0:1, :] + ttf * (type_ref[1:2, :] - type_ref[0:1, :])
    e = rows + pos_ref[...] + type_emb
    mean = jnp.mean(e, axis=1, keepdims=True)
    d = e - mean
    var = jnp.mean(d * d, axis=1, keepdims=True)
    normed = d * lax.rsqrt(var + _EPS)
    out_ref[...] = normed * gamma_ref[...] + beta_ref[...]


def _tc_fuse(rows, pos, tt2, type_table, gamma2, beta2):
    grid = (_SBLK, _B)
    rows_map = lambda i, j: (j * _SBLK + i, 0)
    return pl.pallas_call(
        _tc_fuse_body,
        grid=grid,
        in_specs=[
            pl.BlockSpec((_ROWS_BLK, _HID), rows_map),
            pl.BlockSpec((_ROWS_BLK, _HID), lambda i, j: (i, 0)),
            pl.BlockSpec((1, 1, _ROWS_BLK), lambda i, j: (j * _SBLK + i, 0, 0)),
            pl.BlockSpec((2, _HID), lambda i, j: (0, 0)),
            pl.BlockSpec((1, _HID), lambda i, j: (0, 0)),
            pl.BlockSpec((1, _HID), lambda i, j: (0, 0)),
        ],
        out_specs=pl.BlockSpec((_ROWS_BLK, _HID), rows_map),
        out_shape=jax.ShapeDtypeStruct((_TOK, _HID), jnp.float32),
    )(rows, pos, tt2, type_table, gamma2, beta2)


def kernel(input_ids, token_type_ids, word_table, type_table, gamma, beta):
    idx3 = input_ids.reshape(_NW, _NCH, _CHUNK)
    rows = _sc_gather(word_table, idx3)
    pos = _pos_table()
    ttf = token_type_ids.astype(jnp.float32).reshape(_B * _SBLK, 1, _ROWS_BLK)
    out = _tc_fuse(
        rows, pos, ttf, type_table, gamma.reshape(1, _HID), beta.reshape(1, _HID)
    )
    return out.reshape(_B, _S, _HID)
